# initial kernel scaffold (unmeasured)
import jax
import jax.numpy as jnp
from jax import lax
from jax.experimental import pallas as pl
from jax.experimental.pallas import tpu as pltpu

N_DEV = 16


def kernel(x, w_mat):
    k_full, blk = x.shape
    _, n = w_mat.shape
    assert k_full == N_DEV * blk

    def body(x_ref, w_ref, out_ref, xrow_ref, send_sems, recv_sems):
        my = lax.axis_index("i")

        for j in range(N_DEV):
            @pl.when(my == j)
            def _():
                xrow_ref[:, j * blk:(j + 1) * blk] = x_ref[j * blk:(j + 1) * blk, :]

        for dst in range(N_DEV):
            @pl.when(my != dst)
            def _():
                rdma = pltpu.make_async_remote_copy(
                    src_ref=x_ref.at[pl.ds(dst * blk, blk), :],
                    dst_ref=xrow_ref.at[:, pl.ds(my * blk, blk)],
                    send_sem=send_sems.at[dst],
                    recv_sem=recv_sems.at[my],
                    device_id=(dst,),
                    device_id_type=pl.DeviceIdType.MESH,
                )
                rdma.start()

        for src in range(N_DEV):
            @pl.when(my != src)
            def _():
                recv = pltpu.make_async_remote_copy(
                    src_ref=x_ref.at[pl.ds(src * blk, blk), :],
                    dst_ref=xrow_ref.at[:, pl.ds(src * blk, blk)],
                    send_sem=send_sems.at[src],
                    recv_sem=recv_sems.at[src],
                    device_id=(src,),
                    device_id_type=pl.DeviceIdType.MESH,
                )
                recv.wait_recv()

        out_ref[...] = jnp.dot(
            xrow_ref[...], w_ref[...], preferred_element_type=jnp.float32
        )

        for dst in range(N_DEV):
            @pl.when(my != dst)
            def _():
                send = pltpu.make_async_remote_copy(
                    src_ref=x_ref.at[pl.ds(dst * blk, blk), :],
                    dst_ref=xrow_ref.at[:, pl.ds(my * blk, blk)],
                    send_sem=send_sems.at[dst],
                    recv_sem=recv_sems.at[my],
                    device_id=(dst,),
                    device_id_type=pl.DeviceIdType.MESH,
                )
                send.wait_send()

    return pl.pallas_call(
        body,
        out_shape=jax.ShapeDtypeStruct((blk, n), jnp.float32),
        in_specs=[
            pl.BlockSpec(memory_space=pltpu.VMEM),
            pl.BlockSpec(memory_space=pltpu.VMEM),
        ],
        out_specs=pl.BlockSpec(memory_space=pltpu.VMEM),
        scratch_shapes=[
            pltpu.VMEM((blk, k_full), jnp.float32),
            pltpu.SemaphoreType.DMA((N_DEV,)),
            pltpu.SemaphoreType.DMA((N_DEV,)),
        ],
        compiler_params=pltpu.CompilerParams(collective_id=0),
    )(x, w_mat)


# baseline (device time: 71956 ns/iter reference)
import jax
import jax.numpy as jnp
from jax import lax
from jax.experimental import pallas as pl
from jax.experimental.pallas import tpu as pltpu

N_DEV = 16


def kernel(x, w_mat):
    k_full, blk = x.shape
    _, n = w_mat.shape
    assert k_full == N_DEV * blk

    def body(x_ref, w_ref, out_ref, xrow_ref, send_sems, recv_sems):
        my = lax.axis_index("i")

        for j in range(N_DEV):
            @pl.when(my == j)
            def _():
                xrow_ref[:, j * blk:(j + 1) * blk] = x_ref[j * blk:(j + 1) * blk, :]

        for dst in range(N_DEV):
            @pl.when(my != dst)
            def _():
                rdma = pltpu.make_async_remote_copy(
                    src_ref=x_ref.at[pl.ds(dst * blk, blk), :],
                    dst_ref=xrow_ref.at[:, pl.ds(my * blk, blk)],
                    send_sem=send_sems.at[dst],
                    recv_sem=recv_sems.at[my],
                    device_id=(dst,),
                    device_id_type=pl.DeviceIdType.MESH,
                )
                rdma.start()

        for src in range(N_DEV):
            @pl.when(my != src)
            def _():
                recv = pltpu.make_async_remote_copy(
                    src_ref=x_ref.at[pl.ds(src * blk, blk), :],
                    dst_ref=xrow_ref.at[:, pl.ds(src * blk, blk)],
                    send_sem=send_sems.at[src],
                    recv_sem=recv_sems.at[src],
                    device_id=(src,),
                    device_id_type=pl.DeviceIdType.MESH,
                )
                recv.wait_recv()

        out_ref[...] = jnp.dot(
            xrow_ref[...], w_ref[...], preferred_element_type=jnp.float32
        )

        for dst in range(N_DEV):
            @pl.when(my != dst)
            def _():
                send = pltpu.make_async_remote_copy(
                    src_ref=x_ref.at[pl.ds(dst * blk, blk), :],
                    dst_ref=xrow_ref.at[:, pl.ds(my * blk, blk)],
                    send_sem=send_sems.at[dst],
                    recv_sem=recv_sems.at[my],
                    device_id=(dst,),
                    device_id_type=pl.DeviceIdType.MESH,
                )
                send.wait_send()

    return pl.pallas_call(
        body,
        out_shape=jax.ShapeDtypeStruct((blk, n), jnp.float32),
        in_specs=[
            pl.BlockSpec(memory_space=pltpu.VMEM),
            pl.BlockSpec(memory_space=pltpu.VMEM),
        ],
        out_specs=pl.BlockSpec(memory_space=pltpu.VMEM),
        scratch_shapes=[
            pltpu.VMEM((blk, k_full), jnp.float32),
            pltpu.SemaphoreType.DMA((N_DEV,)),
            pltpu.SemaphoreType.DMA((N_DEV,)),
        ],
        compiler_params=pltpu.CompilerParams(
            vmem_limit_bytes=100 * 1024 * 1024,
        ),
    )(x, w_mat)


# device time: 52170 ns/iter; 1.3793x vs baseline; 1.3793x over previous
import jax
import jax.numpy as jnp
from jax import lax
from jax.experimental import pallas as pl
from jax.experimental.pallas import tpu as pltpu

N_DEV = 16


def kernel(x, w_mat):
    k_full, blk = x.shape
    _, n = w_mat.shape
    assert k_full == N_DEV * blk

    def body(x_ref, w_ref, out_ref, xbf_ref, xrow_ref, send_sems, recv_sems):
        my = lax.axis_index("i")

        xbf_ref[...] = x_ref[...].astype(jnp.bfloat16)

        for j in range(N_DEV):
            @pl.when(my == j)
            def _():
                xrow_ref[:, j * blk:(j + 1) * blk] = xbf_ref[j * blk:(j + 1) * blk, :]

        for dst in range(N_DEV):
            @pl.when(my != dst)
            def _():
                rdma = pltpu.make_async_remote_copy(
                    src_ref=xbf_ref.at[pl.ds(dst * blk, blk), :],
                    dst_ref=xrow_ref.at[:, pl.ds(my * blk, blk)],
                    send_sem=send_sems.at[dst],
                    recv_sem=recv_sems.at[my],
                    device_id=(dst,),
                    device_id_type=pl.DeviceIdType.MESH,
                )
                rdma.start()

        for src in range(N_DEV):
            @pl.when(my != src)
            def _():
                recv = pltpu.make_async_remote_copy(
                    src_ref=xbf_ref.at[pl.ds(src * blk, blk), :],
                    dst_ref=xrow_ref.at[:, pl.ds(src * blk, blk)],
                    send_sem=send_sems.at[src],
                    recv_sem=recv_sems.at[src],
                    device_id=(src,),
                    device_id_type=pl.DeviceIdType.MESH,
                )
                recv.wait_recv()

        out_ref[...] = jnp.dot(
            xrow_ref[...].astype(jnp.float32),
            w_ref[...],
            preferred_element_type=jnp.float32,
        )

        for dst in range(N_DEV):
            @pl.when(my != dst)
            def _():
                send = pltpu.make_async_remote_copy(
                    src_ref=xbf_ref.at[pl.ds(dst * blk, blk), :],
                    dst_ref=xrow_ref.at[:, pl.ds(my * blk, blk)],
                    send_sem=send_sems.at[dst],
                    recv_sem=recv_sems.at[my],
                    device_id=(dst,),
                    device_id_type=pl.DeviceIdType.MESH,
                )
                send.wait_send()

    return pl.pallas_call(
        body,
        out_shape=jax.ShapeDtypeStruct((blk, n), jnp.float32),
        in_specs=[
            pl.BlockSpec(memory_space=pltpu.VMEM),
            pl.BlockSpec(memory_space=pltpu.VMEM),
        ],
        out_specs=pl.BlockSpec(memory_space=pltpu.VMEM),
        scratch_shapes=[
            pltpu.VMEM((k_full, blk), jnp.bfloat16),
            pltpu.VMEM((blk, k_full), jnp.bfloat16),
            pltpu.SemaphoreType.DMA((N_DEV,)),
            pltpu.SemaphoreType.DMA((N_DEV,)),
        ],
        compiler_params=pltpu.CompilerParams(
            vmem_limit_bytes=100 * 1024 * 1024,
        ),
    )(x, w_mat)


# device time: 49248 ns/iter; 1.4611x vs baseline; 1.0593x over previous
import jax
import jax.numpy as jnp
from jax import lax
from jax.experimental import pallas as pl
from jax.experimental.pallas import tpu as pltpu

N_DEV = 16


def kernel(x, w_mat):
    k_full, blk = x.shape
    _, n = w_mat.shape
    assert k_full == N_DEV * blk

    def body(x_ref, w_ref, out_ref, xbf_ref, xrow_ref, send_sems, recv_sems):
        my = lax.axis_index("i")

        xbf_ref[...] = x_ref[...].astype(jnp.bfloat16)

        for off in range(1, N_DEV):
            dst = (my + off) % N_DEV
            rdma = pltpu.make_async_remote_copy(
                src_ref=xbf_ref.at[pl.ds(dst * blk, blk), :],
                dst_ref=xrow_ref.at[:, pl.ds(my * blk, blk)],
                send_sem=send_sems.at[dst],
                recv_sem=recv_sems.at[my],
                device_id=(dst,),
                device_id_type=pl.DeviceIdType.MESH,
            )
            rdma.start()

        out_ref[...] = jnp.dot(
            xbf_ref[pl.ds(my * blk, blk), :].astype(jnp.float32),
            w_ref[pl.ds(my * blk, blk), :],
            preferred_element_type=jnp.float32,
        )

        for off in range(1, N_DEV):
            src = (my - off) % N_DEV
            recv = pltpu.make_async_remote_copy(
                src_ref=xbf_ref.at[pl.ds(src * blk, blk), :],
                dst_ref=xrow_ref.at[:, pl.ds(src * blk, blk)],
                send_sem=send_sems.at[src],
                recv_sem=recv_sems.at[src],
                device_id=(src,),
                device_id_type=pl.DeviceIdType.MESH,
            )
            recv.wait_recv()
            out_ref[...] += jnp.dot(
                xrow_ref[:, pl.ds(src * blk, blk)].astype(jnp.float32),
                w_ref[pl.ds(src * blk, blk), :],
                preferred_element_type=jnp.float32,
            )

        for off in range(1, N_DEV):
            dst = (my + off) % N_DEV
            send = pltpu.make_async_remote_copy(
                src_ref=xbf_ref.at[pl.ds(dst * blk, blk), :],
                dst_ref=xrow_ref.at[:, pl.ds(my * blk, blk)],
                send_sem=send_sems.at[dst],
                recv_sem=recv_sems.at[my],
                device_id=(dst,),
                device_id_type=pl.DeviceIdType.MESH,
            )
            send.wait_send()

    return pl.pallas_call(
        body,
        out_shape=jax.ShapeDtypeStruct((blk, n), jnp.float32),
        in_specs=[
            pl.BlockSpec(memory_space=pltpu.VMEM),
            pl.BlockSpec(memory_space=pltpu.VMEM),
        ],
        out_specs=pl.BlockSpec(memory_space=pltpu.VMEM),
        scratch_shapes=[
            pltpu.VMEM((k_full, blk), jnp.bfloat16),
            pltpu.VMEM((blk, k_full), jnp.bfloat16),
            pltpu.SemaphoreType.DMA((N_DEV,)),
            pltpu.SemaphoreType.DMA((N_DEV,)),
        ],
        compiler_params=pltpu.CompilerParams(
            vmem_limit_bytes=100 * 1024 * 1024,
        ),
    )(x, w_mat)


# device time: 46682 ns/iter; 1.5414x vs baseline; 1.0550x over previous
import jax
import jax.numpy as jnp
from jax import lax
from jax.experimental import pallas as pl
from jax.experimental.pallas import tpu as pltpu

N_DEV = 16


def kernel(x, w_mat):
    k_full, blk = x.shape
    _, n = w_mat.shape
    assert k_full == N_DEV * blk

    def body(x_ref, w_ref, out_ref, xbf_ref, xrow_ref, send_sems, recv_sems,
             bar_sems):
        my = lax.axis_index("i")

        xbf_ref[...] = x_ref[...].astype(jnp.bfloat16)

        bar0 = pltpu.get_barrier_semaphore()
        pl.semaphore_signal(
            bar0, inc=1,
            device_id=((my + 1) % N_DEV,),
            device_id_type=pl.DeviceIdType.MESH,
        )
        pl.semaphore_wait(bar0, 1)
        for r in range(1, 4):
            pl.semaphore_signal(
                bar_sems.at[r - 1], inc=1,
                device_id=((my + (1 << r)) % N_DEV,),
                device_id_type=pl.DeviceIdType.MESH,
            )
            pl.semaphore_wait(bar_sems.at[r - 1], 1)

        for off in range(1, N_DEV):
            dst = (my + off) % N_DEV
            pltpu.make_async_remote_copy(
                src_ref=xbf_ref.at[pl.ds(dst * blk, blk), :],
                dst_ref=xrow_ref.at[:, pl.ds(my * blk, blk)],
                send_sem=send_sems.at[dst],
                recv_sem=recv_sems.at[my],
                device_id=(dst,),
                device_id_type=pl.DeviceIdType.MESH,
            ).start()

        out_ref[...] = jnp.dot(
            xbf_ref[pl.ds(my * blk, blk), :].astype(jnp.float32),
            w_ref[pl.ds(my * blk, blk), :],
            preferred_element_type=jnp.float32,
        )

        for off in range(1, N_DEV):
            src = (my - off) % N_DEV
            pltpu.make_async_remote_copy(
                src_ref=xbf_ref.at[pl.ds(src * blk, blk), :],
                dst_ref=xrow_ref.at[:, pl.ds(src * blk, blk)],
                send_sem=send_sems.at[src],
                recv_sem=recv_sems.at[src],
                device_id=(src,),
                device_id_type=pl.DeviceIdType.MESH,
            ).wait_recv()
            out_ref[...] += jnp.dot(
                xrow_ref[:, pl.ds(src * blk, blk)].astype(jnp.float32),
                w_ref[pl.ds(src * blk, blk), :],
                preferred_element_type=jnp.float32,
            )

        for off in range(1, N_DEV):
            dst = (my + off) % N_DEV
            pltpu.make_async_remote_copy(
                src_ref=xbf_ref.at[pl.ds(dst * blk, blk), :],
                dst_ref=xrow_ref.at[:, pl.ds(my * blk, blk)],
                send_sem=send_sems.at[dst],
                recv_sem=recv_sems.at[my],
                device_id=(dst,),
                device_id_type=pl.DeviceIdType.MESH,
            ).wait_send()

    return pl.pallas_call(
        body,
        out_shape=jax.ShapeDtypeStruct((blk, n), jnp.float32),
        in_specs=[
            pl.BlockSpec(memory_space=pltpu.VMEM),
            pl.BlockSpec(memory_space=pltpu.VMEM),
        ],
        out_specs=pl.BlockSpec(memory_space=pltpu.VMEM),
        scratch_shapes=[
            pltpu.VMEM((k_full, blk), jnp.bfloat16),
            pltpu.VMEM((blk, k_full), jnp.bfloat16),
            pltpu.SemaphoreType.DMA((N_DEV,)),
            pltpu.SemaphoreType.DMA((N_DEV,)),
            pltpu.SemaphoreType.REGULAR((3,)),
        ],
        compiler_params=pltpu.CompilerParams(
            vmem_limit_bytes=100 * 1024 * 1024,
            collective_id=0,
        ),
    )(x, w_mat)


# device time: 42533 ns/iter; 1.6918x vs baseline; 1.0975x over previous
import jax
import jax.numpy as jnp
from jax import lax
from jax.experimental import pallas as pl
from jax.experimental.pallas import tpu as pltpu

N_DEV = 16


def kernel(x, w_mat):
    k_full, blk = x.shape
    _, n = w_mat.shape
    assert k_full == N_DEV * blk

    def body(x_ref, w_ref, out_ref, xbf_ref, xrow_ref, send_sems, recv_sems):
        my = lax.axis_index("i")

        xbf_ref[...] = x_ref[...].astype(jnp.bfloat16)

        bar0 = pltpu.get_barrier_semaphore()
        pl.semaphore_signal(
            bar0, inc=1,
            device_id=(my,),
            device_id_type=pl.DeviceIdType.MESH,
        )
        pl.semaphore_wait(bar0, 1)

        for off in range(1, N_DEV):
            dst = (my + off) % N_DEV
            pltpu.make_async_remote_copy(
                src_ref=xbf_ref.at[pl.ds(dst * blk, blk), :],
                dst_ref=xrow_ref.at[:, pl.ds(my * blk, blk)],
                send_sem=send_sems.at[dst],
                recv_sem=recv_sems.at[my],
                device_id=(dst,),
                device_id_type=pl.DeviceIdType.MESH,
            ).start()

        out_ref[...] = jnp.dot(
            xbf_ref[pl.ds(my * blk, blk), :].astype(jnp.float32),
            w_ref[pl.ds(my * blk, blk), :],
            preferred_element_type=jnp.float32,
        )

        for off in range(1, N_DEV):
            src = (my - off) % N_DEV
            pltpu.make_async_remote_copy(
                src_ref=xbf_ref.at[pl.ds(src * blk, blk), :],
                dst_ref=xrow_ref.at[:, pl.ds(src * blk, blk)],
                send_sem=send_sems.at[src],
                recv_sem=recv_sems.at[src],
                device_id=(src,),
                device_id_type=pl.DeviceIdType.MESH,
            ).wait_recv()
            out_ref[...] += jnp.dot(
                xrow_ref[:, pl.ds(src * blk, blk)].astype(jnp.float32),
                w_ref[pl.ds(src * blk, blk), :],
                preferred_element_type=jnp.float32,
            )

        for off in range(1, N_DEV):
            dst = (my + off) % N_DEV
            pltpu.make_async_remote_copy(
                src_ref=xbf_ref.at[pl.ds(dst * blk, blk), :],
                dst_ref=xrow_ref.at[:, pl.ds(my * blk, blk)],
                send_sem=send_sems.at[dst],
                recv_sem=recv_sems.at[my],
                device_id=(dst,),
                device_id_type=pl.DeviceIdType.MESH,
            ).wait_send()

    return pl.pallas_call(
        body,
        out_shape=jax.ShapeDtypeStruct((blk, n), jnp.float32),
        in_specs=[
            pl.BlockSpec(memory_space=pltpu.VMEM),
            pl.BlockSpec(memory_space=pltpu.VMEM),
        ],
        out_specs=pl.BlockSpec(memory_space=pltpu.VMEM),
        scratch_shapes=[
            pltpu.VMEM((k_full, blk), jnp.bfloat16),
            pltpu.VMEM((blk, k_full), jnp.bfloat16),
            pltpu.SemaphoreType.DMA((N_DEV,)),
            pltpu.SemaphoreType.DMA((N_DEV,)),
        ],
        compiler_params=pltpu.CompilerParams(
            vmem_limit_bytes=100 * 1024 * 1024,
            collective_id=0,
        ),
    )(x, w_mat)
